# baseline (speedup 1.0000x reference)
import functools

import jax
import jax.numpy as jnp
from jax import lax
from jax.experimental import pallas as pl
from jax.experimental.pallas import tpu as pltpu
from jax.experimental.pallas import tpu_sc as plsc

C = 16
G = 128
NC = 2
NS = 16
NW = NC * NS
L = 16
B = 320
GRP = B // L
NIDX = 8 * B
ND = NIDX // 128

_OFFS = [dz * G * G + dy * G + dx
         for dz in (0, 1) for dy in (0, 1) for dx in (0, 1)]

_GDN = lax.GatherDimensionNumbers(
    offset_dims=(), collapsed_slice_dims=(0,), start_index_map=(0,)
)


def _lane_permute(v, la):
    return lax.gather(v, la[:, None], _GDN, (1,),
                      mode=lax.GatherScatterMode.PROMISE_IN_BOUNDS)


def _deinterleave(a, b, c, comp):
    lane = lax.iota(jnp.int32, L)
    pos = 3 * lane + comp
    la = pos & (L - 1)
    ga = _lane_permute(a, la)
    gb = _lane_permute(b, la)
    gc = _lane_permute(c, la)
    ka = (L - comp + 2) // 3
    kb = (2 * L - comp + 2) // 3
    return jnp.where(lane < ka, ga, jnp.where(lane < kb, gb, gc))


def _body(nchunks, xf, table, out, coords, idxb, wb, rows, outb, sem):
    wid = lax.axis_index("s") * NC + lax.axis_index("c")
    my_chunks = (nchunks - 1 - wid) // NW + 1

    def chunk_body(i, carry):
        chunk = wid + i * NW
        off = chunk * B
        pltpu.sync_copy(xf.at[pl.ds(off * 3, 3 * B)], coords.at[pl.ds(0, 3 * B)])

        for g in range(GRP):
            s = g * L
            a = coords[pl.ds(3 * s, L)]
            b = coords[pl.ds(3 * s + L, L)]
            cc = coords[pl.ds(3 * s + 2 * L, L)]
            vx = _deinterleave(a, b, cc, 0)
            vy = _deinterleave(a, b, cc, 1)
            vz = _deinterleave(a, b, cc, 2)
            ix = (vx + 1.0) * (0.5 * (G - 1))
            iy = (vy + 1.0) * (0.5 * (G - 1))
            iz = (vz + 1.0) * (0.5 * (G - 1))
            x0 = jnp.minimum(jnp.maximum(ix.astype(jnp.int32), 0), G - 2)
            y0 = jnp.minimum(jnp.maximum(iy.astype(jnp.int32), 0), G - 2)
            z0 = jnp.minimum(jnp.maximum(iz.astype(jnp.int32), 0), G - 2)
            fx = ix - x0.astype(jnp.float32)
            fy = iy - y0.astype(jnp.float32)
            fz = iz - z0.astype(jnp.float32)
            gx = 1.0 - fx
            gy = 1.0 - fy
            gz = 1.0 - fz
            base = z0 * (G * G) + y0 * G + x0
            wzy = [gz * gy, gz * fy, fz * gy, fz * fy]
            wx = [gx, fx]
            for c in range(8):
                pos = c * B + s
                idxb[pos // 128, pl.ds(pos % 128, L)] = base + _OFFS[c]
                wb[pl.ds(pos, L)] = wzy[c >> 1] * wx[c & 1]

        cps = [
            pltpu.async_copy(
                table.at[idxb.at[j]], rows.at[pl.ds(j * 128, 128)], sem
            )
            for j in range(ND)
        ]
        for cp in cps:
            cp.wait()

        def grp_body(g, c2):
            s = g * L
            wvs = [wb[pl.ds(c * B + s, L)] for c in range(8)]
            for lane in range(L):
                p = s + lane
                acc = wvs[0][lane] * rows[p]
                for c in range(1, 8):
                    acc = acc + wvs[c][lane] * rows[c * B + p]
                outb[p] = acc
            return c2

        lax.fori_loop(0, GRP, grp_body, 0)
        pltpu.sync_copy(outb, out.at[pl.ds(off, B)])
        return carry

    lax.fori_loop(0, my_chunks, chunk_body, 0)


def kernel(x, feature_grid):
    n = x.shape[0]
    assert n % B == 0, n
    nchunks = n // B
    grid = feature_grid[0]
    table = jnp.transpose(grid, (1, 2, 3, 0)).reshape(G * G * G, C)
    xf = x.reshape(-1)

    mesh = plsc.VectorSubcoreMesh(core_axis_name="c", subcore_axis_name="s")
    run = pl.kernel(
        functools.partial(_body, nchunks),
        out_type=jax.ShapeDtypeStruct((n, C), jnp.float32),
        mesh=mesh,
        compiler_params=pltpu.CompilerParams(use_tc_tiling_on_sc=False),
        scratch_types=[
            pltpu.VMEM((3 * B,), jnp.float32),
            pltpu.VMEM((ND, 128), jnp.int32),
            pltpu.VMEM((NIDX,), jnp.float32),
            pltpu.VMEM((NIDX, C), jnp.float32),
            pltpu.VMEM((B, C), jnp.float32),
            pltpu.SemaphoreType.DMA,
        ],
    )
    return run(xf, table)


# speedup vs baseline: 1.8119x; 1.8119x over previous
import functools

import jax
import jax.numpy as jnp
from jax import lax
from jax.experimental import pallas as pl
from jax.experimental.pallas import tpu as pltpu
from jax.experimental.pallas import tpu_sc as plsc

C = 16
G = 128
NC = 2
NS = 16
NW = NC * NS
L = 16
B = 320
GRP = B // L
NIDX = 8 * B
ND = NIDX // 128

NVOX = G * G * G
VPS = NVOX // NS
VC = 1024

_OFFS = [dz * G * G + dy * G + dx
         for dz in (0, 1) for dy in (0, 1) for dx in (0, 1)]

_GDN = lax.GatherDimensionNumbers(
    offset_dims=(), collapsed_slice_dims=(0,), start_index_map=(0,)
)


def _lane_permute(v, la):
    return lax.gather(v, la[:, None], _GDN, (1,),
                      mode=lax.GatherScatterMode.PROMISE_IN_BOUNDS)


def _transpose16(regs):
    lane = lax.iota(jnp.int32, L)
    r = list(regs)
    for k in (1, 2, 4, 8):
        keep = (lane & k) == 0
        perm = lane ^ k
        for i in range(L):
            if i & k:
                continue
            a, b = r[i], r[i | k]
            a_s = _lane_permute(a, perm)
            b_s = _lane_permute(b, perm)
            r[i] = jnp.where(keep, a, b_s)
            r[i | k] = jnp.where(keep, a_s, b)
    return r


def _body(nchunks, n, xf, gridf, out, tbl,
          coords, idxb, wb, rows, outb, chbuf, obuf, sem):
    scid = lax.axis_index("c")
    sid = lax.axis_index("s")
    wid = sid * NC + scid
    tb = scid * NVOX

    def fmt_chunk(k, carry):
        vbase = sid * VPS + k * VC
        for c in range(C):
            pltpu.sync_copy(gridf.at[pl.ds(c * NVOX + vbase, VC)],
                            chbuf.at[pl.ds(c * VC, VC)])

        def blk_body(blk, c2):
            regs = [chbuf[pl.ds(c * VC + blk * L, L)] for c in range(C)]
            outs = _transpose16(regs)
            for j in range(L):
                obuf[blk * L + j] = outs[j]
            return c2

        lax.fori_loop(0, VC // L, blk_body, 0)
        pltpu.sync_copy(obuf, tbl.at[pl.ds(tb + vbase, VC)])
        return carry

    lax.fori_loop(0, VPS // VC, fmt_chunk, 0)
    plsc.subcore_barrier()

    def chunk_body(i, carry):
        chunk = wid + i * NW
        off = chunk * B
        pltpu.sync_copy(xf.at[pl.ds(off, B)], coords.at[pl.ds(0, B)])
        pltpu.sync_copy(xf.at[pl.ds(n + off, B)], coords.at[pl.ds(B, B)])
        pltpu.sync_copy(xf.at[pl.ds(2 * n + off, B)], coords.at[pl.ds(2 * B, B)])

        for g in range(GRP):
            s = g * L
            vx = coords[pl.ds(s, L)]
            vy = coords[pl.ds(B + s, L)]
            vz = coords[pl.ds(2 * B + s, L)]
            ix = (vx + 1.0) * (0.5 * (G - 1))
            iy = (vy + 1.0) * (0.5 * (G - 1))
            iz = (vz + 1.0) * (0.5 * (G - 1))
            x0 = jnp.minimum(jnp.maximum(ix.astype(jnp.int32), 0), G - 2)
            y0 = jnp.minimum(jnp.maximum(iy.astype(jnp.int32), 0), G - 2)
            z0 = jnp.minimum(jnp.maximum(iz.astype(jnp.int32), 0), G - 2)
            fx = ix - x0.astype(jnp.float32)
            fy = iy - y0.astype(jnp.float32)
            fz = iz - z0.astype(jnp.float32)
            gx = 1.0 - fx
            gy = 1.0 - fy
            gz = 1.0 - fz
            base = tb + z0 * (G * G) + y0 * G + x0
            wzy = [gz * gy, gz * fy, fz * gy, fz * fy]
            wx = [gx, fx]
            for c in range(8):
                pos = c * B + s
                idxb[pos // 128, pl.ds(pos % 128, L)] = base + _OFFS[c]
                wb[pl.ds(pos, L)] = wzy[c >> 1] * wx[c & 1]

        cps = [
            pltpu.async_copy(
                tbl.at[idxb.at[j]], rows.at[pl.ds(j * 128, 128)], sem
            )
            for j in range(ND)
        ]
        for cp in cps:
            cp.wait()

        def grp_body(g, c2):
            s = g * L
            wvs = [wb[pl.ds(c * B + s, L)] for c in range(8)]
            for lane in range(L):
                p = s + lane
                acc = wvs[0][lane] * rows[p]
                for c in range(1, 8):
                    acc = acc + wvs[c][lane] * rows[c * B + p]
                outb[p] = acc
            return c2

        lax.fori_loop(0, GRP, grp_body, 0)
        pltpu.sync_copy(outb, out.at[pl.ds(off, B)])
        return carry

    lax.fori_loop(0, (nchunks - 1 - wid) // NW + 1, chunk_body, 0)


def kernel(x, feature_grid):
    n = x.shape[0]
    assert n % B == 0, n
    nchunks = n // B
    gridf = feature_grid.reshape(-1)
    xf = x.T.reshape(-1)

    mesh = plsc.VectorSubcoreMesh(core_axis_name="c", subcore_axis_name="s")
    run = pl.kernel(
        functools.partial(_body, nchunks, n),
        out_type=(
            jax.ShapeDtypeStruct((n, C), jnp.float32),
            jax.ShapeDtypeStruct((NC * NVOX, C), jnp.float32),
        ),
        mesh=mesh,
        compiler_params=pltpu.CompilerParams(use_tc_tiling_on_sc=False),
        scratch_types=[
            pltpu.VMEM((3 * B,), jnp.float32),
            pltpu.VMEM((ND, 128), jnp.int32),
            pltpu.VMEM((NIDX,), jnp.float32),
            pltpu.VMEM((NIDX, C), jnp.float32),
            pltpu.VMEM((B, C), jnp.float32),
            pltpu.VMEM((C * VC,), jnp.float32),
            pltpu.VMEM((VC, C), jnp.float32),
            pltpu.SemaphoreType.DMA,
        ],
    )
    outp, _ = run(xf, gridf)
    return outp


# speedup vs baseline: 2.8719x; 1.5850x over previous
import functools

import jax
import jax.numpy as jnp
from jax import lax
from jax.experimental import pallas as pl
from jax.experimental.pallas import tpu as pltpu
from jax.experimental.pallas import tpu_sc as plsc

C = 16
G = 128
NC = 2
NS = 16
NW = NC * NS
L = 16
B = 320
GRP = B // L
NIDX = 8 * B
ND = NIDX // 128

NVOX = G * G * G
VPS = NVOX // NS
VC = 1024

_OFFS = [dz * G * G + dy * G + dx
         for dz in (0, 1) for dy in (0, 1) for dx in (0, 1)]

_GDN = lax.GatherDimensionNumbers(
    offset_dims=(), collapsed_slice_dims=(0,), start_index_map=(0,)
)


def _lane_permute(v, la):
    return lax.gather(v, la[:, None], _GDN, (1,),
                      mode=lax.GatherScatterMode.PROMISE_IN_BOUNDS)


def _transpose16(regs):
    lane = lax.iota(jnp.int32, L)
    r = list(regs)
    for k in (1, 2, 4, 8):
        keep = (lane & k) == 0
        perm = lane ^ k
        for i in range(L):
            if i & k:
                continue
            a, b = r[i], r[i | k]
            a_s = _lane_permute(a, perm)
            b_s = _lane_permute(b, perm)
            r[i] = jnp.where(keep, a, b_s)
            r[i | k] = jnp.where(keep, a_s, b)
    return r


def _body(nchunks, n, xf, gridf, out, tbl,
          coords, idxb, wb, rows, outb, chbuf, obuf, sem):
    scid = lax.axis_index("c")
    sid = lax.axis_index("s")
    wid = sid * NC + scid
    tb = scid * NVOX

    def fmt_chunk(k, carry):
        vbase = sid * VPS + k * VC
        pltpu.sync_copy(gridf.at[:, pl.ds(vbase, VC)], chbuf)

        def blk_body(blk, c2):
            regs = [chbuf[c, pl.ds(blk * L, L)] for c in range(C)]
            outs = _transpose16(regs)
            for j in range(L):
                obuf[blk * L + j] = outs[j]
            return c2

        lax.fori_loop(0, VC // L, blk_body, 0)
        pltpu.sync_copy(obuf, tbl.at[pl.ds(tb + vbase, VC)])
        return carry

    lax.fori_loop(0, VPS // VC, fmt_chunk, 0)
    plsc.subcore_barrier()

    def chunk_body(i, carry):
        chunk = wid + i * NW
        off = chunk * B
        pltpu.sync_copy(xf.at[pl.ds(off, B)], coords.at[pl.ds(0, B)])
        pltpu.sync_copy(xf.at[pl.ds(n + off, B)], coords.at[pl.ds(B, B)])
        pltpu.sync_copy(xf.at[pl.ds(2 * n + off, B)], coords.at[pl.ds(2 * B, B)])

        for g in range(GRP):
            s = g * L
            vx = coords[pl.ds(s, L)]
            vy = coords[pl.ds(B + s, L)]
            vz = coords[pl.ds(2 * B + s, L)]
            ix = (vx + 1.0) * (0.5 * (G - 1))
            iy = (vy + 1.0) * (0.5 * (G - 1))
            iz = (vz + 1.0) * (0.5 * (G - 1))
            x0 = jnp.minimum(jnp.maximum(ix.astype(jnp.int32), 0), G - 2)
            y0 = jnp.minimum(jnp.maximum(iy.astype(jnp.int32), 0), G - 2)
            z0 = jnp.minimum(jnp.maximum(iz.astype(jnp.int32), 0), G - 2)
            fx = ix - x0.astype(jnp.float32)
            fy = iy - y0.astype(jnp.float32)
            fz = iz - z0.astype(jnp.float32)
            gx = 1.0 - fx
            gy = 1.0 - fy
            gz = 1.0 - fz
            base = tb + z0 * (G * G) + y0 * G + x0
            wzy = [gz * gy, gz * fy, fz * gy, fz * fy]
            wx = [gx, fx]
            for c in range(8):
                pos = c * B + s
                idxb[pos // 128, pl.ds(pos % 128, L)] = base + _OFFS[c]
                wb[pl.ds(pos, L)] = wzy[c >> 1] * wx[c & 1]

        cps = [
            pltpu.async_copy(
                tbl.at[idxb.at[j]], rows.at[pl.ds(j * 128, 128)], sem
            )
            for j in range(ND)
        ]
        for cp in cps:
            cp.wait()

        def grp_body(g, c2):
            s = g * L
            wvs = [wb[pl.ds(c * B + s, L)] for c in range(8)]
            for lane in range(L):
                p = s + lane
                acc = wvs[0][lane] * rows[p]
                for c in range(1, 8):
                    acc = acc + wvs[c][lane] * rows[c * B + p]
                outb[p] = acc
            return c2

        lax.fori_loop(0, GRP, grp_body, 0)
        pltpu.sync_copy(outb, out.at[pl.ds(off, B)])
        return carry

    lax.fori_loop(0, (nchunks - 1 - wid) // NW + 1, chunk_body, 0)


def kernel(x, feature_grid):
    n = x.shape[0]
    assert n % B == 0, n
    nchunks = n // B
    gridf = feature_grid.reshape(C, NVOX)
    xf = x.T.reshape(-1)

    mesh = plsc.VectorSubcoreMesh(core_axis_name="c", subcore_axis_name="s")
    run = pl.kernel(
        functools.partial(_body, nchunks, n),
        out_type=(
            jax.ShapeDtypeStruct((n, C), jnp.float32),
            jax.ShapeDtypeStruct((NC * NVOX, C), jnp.float32),
        ),
        mesh=mesh,
        compiler_params=pltpu.CompilerParams(use_tc_tiling_on_sc=False),
        scratch_types=[
            pltpu.VMEM((3 * B,), jnp.float32),
            pltpu.VMEM((ND, 128), jnp.int32),
            pltpu.VMEM((NIDX,), jnp.float32),
            pltpu.VMEM((NIDX, C), jnp.float32),
            pltpu.VMEM((B, C), jnp.float32),
            pltpu.VMEM((C, VC), jnp.float32),
            pltpu.VMEM((VC, C), jnp.float32),
            pltpu.SemaphoreType.DMA,
        ],
    )
    outp, _ = run(xf, gridf)
    return outp


# speedup vs baseline: 3.5285x; 1.2286x over previous
import functools

import jax
import jax.numpy as jnp
from jax import lax
from jax.experimental import pallas as pl
from jax.experimental.pallas import tpu as pltpu
from jax.experimental.pallas import tpu_sc as plsc

C = 16
G = 128
NC = 2
NS = 16
NW = NC * NS
L = 16
B = 320
GRP = B // L
NIDX = 8 * B
ND = NIDX // 128

NVOX = G * G * G
VPS = NVOX // NS
VC = 512

_OFFS = [dz * G * G + dy * G + dx
         for dz in (0, 1) for dy in (0, 1) for dx in (0, 1)]

_GDN = lax.GatherDimensionNumbers(
    offset_dims=(), collapsed_slice_dims=(0,), start_index_map=(0,)
)


def _lane_permute(v, la):
    return lax.gather(v, la[:, None], _GDN, (1,),
                      mode=lax.GatherScatterMode.PROMISE_IN_BOUNDS)


def _transpose16(regs):
    lane = lax.iota(jnp.int32, L)
    r = list(regs)
    for k in (1, 2, 4, 8):
        keep = (lane & k) == 0
        perm = lane ^ k
        for i in range(L):
            if i & k:
                continue
            a, b = r[i], r[i | k]
            a_s = _lane_permute(a, perm)
            b_s = _lane_permute(b, perm)
            r[i] = jnp.where(keep, a, b_s)
            r[i | k] = jnp.where(keep, a_s, b)
    return r


def _body(nchunks, n, xf, gridf, out, tbl,
          coords, idxb, wb, rows, outb, chbuf, obuf, sem, sem_in, sem_out):
    scid = lax.axis_index("c")
    sid = lax.axis_index("s")
    wid = sid * NC + scid
    tb = scid * NVOX

    NFC = VPS // VC
    fv0 = sid * VPS

    def fmt_in(k, b):
        return pltpu.make_async_copy(
            gridf.at[:, pl.ds(fv0 + k * VC, VC)], chbuf.at[b], sem_in
        )

    def fmt_out(k, b):
        return pltpu.make_async_copy(
            obuf.at[b], tbl.at[pl.ds(tb + fv0 + k * VC, VC)], sem_out
        )

    fmt_in(0, 0).start()

    def fmt_chunk2(k2, carry):
        for b in (0, 1):
            k = k2 * 2 + b

            @pl.when(k + 1 < NFC)
            def _():
                fmt_in(k + 1, 1 - b).start()

            fmt_in(k, b).wait()

            @pl.when(k >= 2)
            def _():
                fmt_out(k - 2, b).wait()

            def blk_body(blk, c2):
                regs = [chbuf[b, c, pl.ds(blk * L, L)] for c in range(C)]
                outs = _transpose16(regs)
                for j in range(L):
                    obuf[b, blk * L + j] = outs[j]
                return c2

            lax.fori_loop(0, VC // L, blk_body, 0)
            fmt_out(k, b).start()
        return carry

    lax.fori_loop(0, NFC // 2, fmt_chunk2, 0)
    fmt_out(NFC - 2, 0).wait()
    fmt_out(NFC - 1, 1).wait()
    plsc.subcore_barrier()

    def chunk_body(i, carry):
        chunk = wid + i * NW
        off = chunk * B
        pltpu.sync_copy(xf.at[:, pl.ds(off, B)], coords)

        for g in range(GRP):
            s = g * L
            vx = coords[0, pl.ds(s, L)]
            vy = coords[1, pl.ds(s, L)]
            vz = coords[2, pl.ds(s, L)]
            ix = (vx + 1.0) * (0.5 * (G - 1))
            iy = (vy + 1.0) * (0.5 * (G - 1))
            iz = (vz + 1.0) * (0.5 * (G - 1))
            x0 = jnp.minimum(jnp.maximum(ix.astype(jnp.int32), 0), G - 2)
            y0 = jnp.minimum(jnp.maximum(iy.astype(jnp.int32), 0), G - 2)
            z0 = jnp.minimum(jnp.maximum(iz.astype(jnp.int32), 0), G - 2)
            fx = ix - x0.astype(jnp.float32)
            fy = iy - y0.astype(jnp.float32)
            fz = iz - z0.astype(jnp.float32)
            gx = 1.0 - fx
            gy = 1.0 - fy
            gz = 1.0 - fz
            base = tb + z0 * (G * G) + y0 * G + x0
            wzy = [gz * gy, gz * fy, fz * gy, fz * fy]
            wx = [gx, fx]
            for c in range(8):
                pos = c * B + s
                idxb[pos // 128, pl.ds(pos % 128, L)] = base + _OFFS[c]
                wb[pl.ds(pos, L)] = wzy[c >> 1] * wx[c & 1]

        cps = [
            pltpu.async_copy(
                tbl.at[idxb.at[j]], rows.at[pl.ds(j * 128, 128)], sem
            )
            for j in range(ND)
        ]
        for cp in cps:
            cp.wait()

        def grp_body(g, c2):
            s = g * L
            wvs = [wb[pl.ds(c * B + s, L)] for c in range(8)]
            for lane in range(L):
                p = s + lane
                acc = wvs[0][lane] * rows[p]
                for c in range(1, 8):
                    acc = acc + wvs[c][lane] * rows[c * B + p]
                outb[p] = acc
            return c2

        lax.fori_loop(0, GRP, grp_body, 0)
        pltpu.sync_copy(outb, out.at[pl.ds(off, B)])
        return carry

    lax.fori_loop(0, (nchunks - 1 - wid) // NW + 1, chunk_body, 0)


def kernel(x, feature_grid):
    n = x.shape[0]
    assert n % B == 0, n
    nchunks = n // B
    gridf = feature_grid.reshape(C, NVOX)
    xf = x.T

    mesh = plsc.VectorSubcoreMesh(core_axis_name="c", subcore_axis_name="s")
    run = pl.kernel(
        functools.partial(_body, nchunks, n),
        out_type=(
            jax.ShapeDtypeStruct((n, C), jnp.float32),
            jax.ShapeDtypeStruct((NC * NVOX, C), jnp.float32),
        ),
        mesh=mesh,
        compiler_params=pltpu.CompilerParams(use_tc_tiling_on_sc=False),
        scratch_types=[
            pltpu.VMEM((3, B), jnp.float32),
            pltpu.VMEM((ND, 128), jnp.int32),
            pltpu.VMEM((NIDX,), jnp.float32),
            pltpu.VMEM((NIDX, C), jnp.float32),
            pltpu.VMEM((B, C), jnp.float32),
            pltpu.VMEM((2, C, VC), jnp.float32),
            pltpu.VMEM((2, VC, C), jnp.float32),
            pltpu.SemaphoreType.DMA,
            pltpu.SemaphoreType.DMA,
            pltpu.SemaphoreType.DMA,
        ],
    )
    outp, _ = run(xf, gridf)
    return outp


# speedup vs baseline: 4.0985x; 1.1616x over previous
import functools

import jax
import jax.numpy as jnp
from jax import lax
from jax.experimental import pallas as pl
from jax.experimental.pallas import tpu as pltpu
from jax.experimental.pallas import tpu_sc as plsc

C = 16
G = 128
NC = 2
NS = 16
NW = NC * NS
L = 16
B = 320
GRP = B // L
NIDX = 8 * B
ND = NIDX // 128

NVOX = G * G * G
VPS = NVOX // NS
VC = 256

_OFFS = [dz * G * G + dy * G + dx
         for dz in (0, 1) for dy in (0, 1) for dx in (0, 1)]

_GDN = lax.GatherDimensionNumbers(
    offset_dims=(), collapsed_slice_dims=(0,), start_index_map=(0,)
)


def _lane_permute(v, la):
    return lax.gather(v, la[:, None], _GDN, (1,),
                      mode=lax.GatherScatterMode.PROMISE_IN_BOUNDS)


def _transpose16(regs):
    lane = lax.iota(jnp.int32, L)
    r = list(regs)
    for k in (1, 2, 4, 8):
        keep = (lane & k) == 0
        perm = lane ^ k
        for i in range(L):
            if i & k:
                continue
            a, b = r[i], r[i | k]
            a_s = _lane_permute(a, perm)
            b_s = _lane_permute(b, perm)
            r[i] = jnp.where(keep, a, b_s)
            r[i | k] = jnp.where(keep, a_s, b)
    return r


def _body(nchunks, n, xf, gridf, out, tbl,
          coords, idxb, wb, rows, outb, chbuf, obuf, sem, sem_in, sem_out, sem_c):
    scid = lax.axis_index("c")
    sid = lax.axis_index("s")
    wid = sid * NC + scid
    tb = scid * NVOX

    NFC = VPS // VC
    fv0 = sid * VPS

    def fmt_in(k, b):
        return pltpu.make_async_copy(
            gridf.at[:, pl.ds(fv0 + k * VC, VC)], chbuf.at[b], sem_in
        )

    def fmt_out(k, b):
        return pltpu.make_async_copy(
            obuf.at[b], tbl.at[pl.ds(tb + fv0 + k * VC, VC)], sem_out
        )

    fmt_in(0, 0).start()

    def fmt_chunk2(k2, carry):
        for b in (0, 1):
            k = k2 * 2 + b

            @pl.when(k + 1 < NFC)
            def _():
                fmt_in(k + 1, 1 - b).start()

            fmt_in(k, b).wait()

            @pl.when(k >= 2)
            def _():
                fmt_out(k - 2, b).wait()

            def blk_body(blk, c2):
                regs = [chbuf[b, c, pl.ds(blk * L, L)] for c in range(C)]
                outs = _transpose16(regs)
                for j in range(L):
                    obuf[b, blk * L + j] = outs[j]
                return c2

            lax.fori_loop(0, VC // L, blk_body, 0)
            fmt_out(k, b).start()
        return carry

    lax.fori_loop(0, NFC // 2, fmt_chunk2, 0)
    fmt_out(NFC - 2, 0).wait()
    fmt_out(NFC - 1, 1).wait()
    plsc.subcore_barrier()

    my = (nchunks - 1 - wid) // NW + 1

    def coord_cp(k, b):
        off = (wid + k * NW) * B
        return pltpu.make_async_copy(
            xf.at[:, pl.ds(off, B)], coords.at[b], sem_c
        )

    def gather_cp(j, b):
        return pltpu.make_async_copy(
            tbl.at[idxb.at[b, j]], rows.at[b, pl.ds(j * 128, 128)], sem
        )

    coord_cp(0, 0).start()

    def build_fire(k, b):
        for g in range(GRP):
            s = g * L
            vx = coords[b, 0, pl.ds(s, L)]
            vy = coords[b, 1, pl.ds(s, L)]
            vz = coords[b, 2, pl.ds(s, L)]
            ix = (vx + 1.0) * (0.5 * (G - 1))
            iy = (vy + 1.0) * (0.5 * (G - 1))
            iz = (vz + 1.0) * (0.5 * (G - 1))
            x0 = jnp.minimum(jnp.maximum(ix.astype(jnp.int32), 0), G - 2)
            y0 = jnp.minimum(jnp.maximum(iy.astype(jnp.int32), 0), G - 2)
            z0 = jnp.minimum(jnp.maximum(iz.astype(jnp.int32), 0), G - 2)
            fx = ix - x0.astype(jnp.float32)
            fy = iy - y0.astype(jnp.float32)
            fz = iz - z0.astype(jnp.float32)
            gx = 1.0 - fx
            gy = 1.0 - fy
            gz = 1.0 - fz
            base = tb + z0 * (G * G) + y0 * G + x0
            wzy = [gz * gy, gz * fy, fz * gy, fz * fy]
            wx = [gx, fx]
            for c in range(8):
                pos = c * B + s
                idxb[b, pos // 128, pl.ds(pos % 128, L)] = base + _OFFS[c]
                wb[b, pl.ds(pos, L)] = wzy[c >> 1] * wx[c & 1]
        for j in range(ND):
            gather_cp(j, b).start()

    def combine_store(k, b):
        for j in range(ND):
            gather_cp(j, b).wait()

        def grp_body(g, c2):
            s = g * L
            wvs = [wb[b, pl.ds(c * B + s, L)] for c in range(8)]
            for lane in range(L):
                p = s + lane
                acc = wvs[0][lane] * rows[b, p]
                for c in range(1, 8):
                    acc = acc + wvs[c][lane] * rows[b, c * B + p]
                outb[p] = acc
            return c2

        lax.fori_loop(0, GRP, grp_body, 0)
        pltpu.sync_copy(outb, out.at[pl.ds((wid + k * NW) * B, B)])

    def pipe_step2(k2, carry):
        for b in (0, 1):
            k = k2 * 2 + b

            @pl.when(k < my)
            def _():
                coord_cp(k, b).wait()

                @pl.when(k + 1 < my)
                def _():
                    coord_cp(k + 1, 1 - b).start()

                build_fire(k, b)

            @pl.when((k >= 1) & (k <= my))
            def _():
                combine_store(k - 1, 1 - b)
        return carry

    lax.fori_loop(0, (my + 2) // 2, pipe_step2, 0)


def kernel(x, feature_grid):
    n = x.shape[0]
    assert n % B == 0, n
    nchunks = n // B
    gridf = feature_grid.reshape(C, NVOX)
    xf = x.T

    mesh = plsc.VectorSubcoreMesh(core_axis_name="c", subcore_axis_name="s")
    run = pl.kernel(
        functools.partial(_body, nchunks, n),
        out_type=(
            jax.ShapeDtypeStruct((n, C), jnp.float32),
            jax.ShapeDtypeStruct((NC * NVOX, C), jnp.float32),
        ),
        mesh=mesh,
        compiler_params=pltpu.CompilerParams(use_tc_tiling_on_sc=False),
        scratch_types=[
            pltpu.VMEM((2, 3, B), jnp.float32),
            pltpu.VMEM((2, ND, 128), jnp.int32),
            pltpu.VMEM((2, NIDX), jnp.float32),
            pltpu.VMEM((2, NIDX, C), jnp.float32),
            pltpu.VMEM((B, C), jnp.float32),
            pltpu.VMEM((2, C, VC), jnp.float32),
            pltpu.VMEM((2, VC, C), jnp.float32),
            pltpu.SemaphoreType.DMA,
            pltpu.SemaphoreType.DMA,
            pltpu.SemaphoreType.DMA,
            pltpu.SemaphoreType.DMA,
        ],
    )
    outp, _ = run(xf, gridf)
    return outp
